# baseline (speedup 1.0000x reference)
import functools

import jax
import jax.numpy as jnp
from jax import lax
from jax.experimental import pallas as pl
from jax.experimental.pallas import tpu as pltpu
from jax.experimental.pallas import tpu_sc as plsc

_D = 256
_NC, _NS = 2, 16
_NW = _NC * _NS
_CH = 256


def _emb_body(nchunk, ids_hbm, table_hbm, out_hbm, idx_v, rows_v, sem):
    wid = lax.axis_index("s") * _NC + lax.axis_index("c")
    base = wid * (nchunk * _CH)
    for t in range(nchunk):
        off = base + t * _CH
        pltpu.sync_copy(ids_hbm.at[pl.ds(off, _CH)], idx_v)
        pltpu.async_copy(table_hbm.at[idx_v], rows_v, sem).wait()
        pltpu.sync_copy(rows_v, out_hbm.at[pl.ds(off, _CH)])


@functools.partial(jax.jit, static_argnums=(2,))
def _emb(flat_ids, table, n):
    nchunk = n // (_NW * _CH)
    grid_kernel = functools.partial(
        pl.kernel,
        out_type=jax.ShapeDtypeStruct((n, _D), jnp.float32),
        mesh=plsc.VectorSubcoreMesh(core_axis_name="c", subcore_axis_name="s"),
        scratch_types=[
            pltpu.VMEM((_CH,), jnp.int32),
            pltpu.VMEM((_CH, _D), jnp.float32),
            pltpu.SemaphoreType.DMA,
        ],
    )
    return grid_kernel(functools.partial(_emb_body, nchunk))(flat_ids, table)


def kernel(input_ids, table):
    n = input_ids.size
    flat = input_ids.reshape((n,))
    out = _emb(flat, table, n)
    return out.reshape(input_ids.shape + (table.shape[1],))
